# baseline (speedup 1.0000x reference)
import functools

import jax
import jax.numpy as jnp
from jax import lax
from jax.experimental import pallas as pl
from jax.experimental.pallas import tpu as pltpu
from jax.experimental.pallas import tpu_sc as plsc

N = 1_000_000
B = 2048
PN = 1 << 20
PAD_NODES = PN - N
R = 8192
C = 512
RB = 512
NPR = 128
CLIP = 27.631021115928547


def _dense_body(p_ref, t_ref, tb_ref, f_ref, a_ref, err_ref, bs_ref, ms_ref,
                acc_ref):
    i = pl.program_id(0)
    p = p_ref[...]
    t = t_ref[...]
    f = f_ref[...].astype(jnp.float32)
    d = p - t
    err = f * d * d
    sel = (lax.broadcasted_iota(jnp.int32, (C, NPR), 0) // 4
           == lax.broadcasted_iota(jnp.int32, (C, NPR), 1)).astype(jnp.float32)
    err_ref[...] = lax.dot_general(err, sel, (((1,), (0,)), ((), ())),
                                   precision=lax.Precision.HIGHEST,
                                   preferred_element_type=jnp.float32)
    a = a_ref[...].astype(jnp.float32)
    tb = tb_ref[...]
    sp_abs = jnp.log(1.0 + jnp.exp(-jnp.abs(p)))
    sp_pos = jnp.maximum(p, 0.0) + sp_abs
    sp_neg = sp_pos - p
    bce = jnp.where(tb, jnp.minimum(sp_neg, CLIP), jnp.minimum(sp_pos, CLIP))
    bs = jnp.sum(bce * a)
    ms = jnp.sum(a)

    @pl.when(i == 0)
    def _zero():
        acc_ref[0] = 0.0
        acc_ref[1] = 0.0

    acc_ref[0] += bs
    acc_ref[1] += ms

    @pl.when(i == pl.num_programs(0) - 1)
    def _emit():
        bs_ref[...] = jnp.full((8, 128), acc_ref[0], jnp.float32)
        ms_ref[...] = jnp.full((8, 128), acc_ref[1], jnp.float32)


def _dense_call(p4, t4, tb4, f4, a4):
    grid = R // RB
    in_spec = pl.BlockSpec((RB, C), lambda i: (i, 0))
    return pl.pallas_call(
        _dense_body,
        grid=(grid,),
        in_specs=[in_spec] * 5,
        out_specs=[
            pl.BlockSpec((RB, NPR), lambda i: (i, 0)),
            pl.BlockSpec((8, 128), lambda i: (0, 0)),
            pl.BlockSpec((8, 128), lambda i: (0, 0)),
        ],
        out_shape=[
            jax.ShapeDtypeStruct((R, NPR), jnp.float32),
            jax.ShapeDtypeStruct((8, 128), jnp.float32),
            jax.ShapeDtypeStruct((8, 128), jnp.float32),
        ],
        scratch_shapes=[pltpu.SMEM((2,), jnp.float32)],
    )(p4, t4, tb4, f4, a4)


SC_TILES = 16
NPT = PN // SC_TILES
PZ = 8192
NPIECES = NPT // PZ
BPT = B // SC_TILES
ACCW = 17
AW = B * ACCW
FW = BPT * ACCW


def _seg_body(err_hbm, ids_hbm, zeros_hbm, initc_hbm, sb_hbm, mb_hbm,
              out_hbm, ebuf, ibuf, acc, cnt, sh, fin_a, fin_c,
              tmp_a, tmp_c, outb, sbv, mbv):
    wid = lax.axis_index("s")
    base = wid * NPT
    lane = lax.iota(jnp.int32, 16)
    ones = jnp.ones((16,), jnp.float32)

    pltpu.sync_copy(zeros_hbm, acc)

    @pl.when(wid == 0)
    def _init0():
        pltpu.sync_copy(initc_hbm, cnt)

    @pl.when(wid != 0)
    def _initn():
        pltpu.sync_copy(zeros_hbm, cnt)

    pltpu.sync_copy(sb_hbm, sbv)
    pltpu.sync_copy(mb_hbm, mbv)

    for pc in range(NPIECES):
        off = base + pc * PZ
        pltpu.sync_copy(err_hbm.at[pl.ds(off, PZ)], ebuf)
        pltpu.sync_copy(ids_hbm.at[pl.ds(off, PZ)], ibuf)

        def _step(v, carry):
            ev = ebuf[pl.ds(v * 16, 16)]
            iv = ibuf[pl.ds(v * 16, 16)]
            ix = iv * ACCW + lane
            plsc.addupdate_scatter(acc, [ix], ev)
            plsc.addupdate_scatter(cnt, [ix], ones)
            return carry

        lax.fori_loop(0, PZ // 16, _step, 0)

    off0 = wid * FW

    def _merge(src, fin, tmp):
        pltpu.sync_copy(src, sh.at[wid])
        plsc.subcore_barrier()
        pltpu.sync_copy(sh.at[0, pl.ds(off0, FW)], fin)
        for s in range(1, SC_TILES):
            pltpu.sync_copy(sh.at[s, pl.ds(off0, FW)], tmp)

            @plsc.parallel_loop(0, FW // 16, 1, unroll=4)
            def _addv(v):
                sl = pl.ds(v * 16, 16)
                fin[sl] = fin[sl] + tmp[sl]
        plsc.subcore_barrier()

    _merge(acc, fin_a, tmp_a)
    _merge(cnt, fin_c, tmp_c)

    sbce = sbv[...] / jnp.maximum(mbv[...], 1.0)
    for g in range(BPT // 16):
        rows = (g * 16 + lane) * ACCW
        sv = jnp.zeros((16,), jnp.float32)
        cv = jnp.zeros((16,), jnp.float32)
        for c in range(16):
            sv = sv + plsc.load_gather(fin_a, [rows + c])
            cv = cv + plsc.load_gather(fin_c, [rows + c])
        outb[pl.ds(g * 16, 16)] = sv / jnp.maximum(cv, 1.0) * (1.0 / 3.0) + sbce
    pltpu.sync_copy(outb, out_hbm.at[pl.ds(wid * BPT, BPT)])


def _seg_call(err_lin, ids_pad, zeros1d, initc1d, sb16, mb16):
    mesh = plsc.VectorSubcoreMesh(core_axis_name="c", subcore_axis_name="s",
                                  num_cores=1, num_subcores=SC_TILES)
    f = functools.partial(
        pl.kernel,
        out_type=jax.ShapeDtypeStruct((B,), jnp.float32),
        mesh=mesh,
        scratch_types=[
            pltpu.VMEM((PZ,), jnp.float32),
            pltpu.VMEM((PZ,), jnp.int32),
            pltpu.VMEM((AW,), jnp.float32),
            pltpu.VMEM((AW,), jnp.float32),
            pltpu.VMEM_SHARED((SC_TILES, AW), jnp.float32),
            pltpu.VMEM((FW,), jnp.float32),
            pltpu.VMEM((FW,), jnp.float32),
            pltpu.VMEM((FW,), jnp.float32),
            pltpu.VMEM((FW,), jnp.float32),
            pltpu.VMEM((BPT,), jnp.float32),
            pltpu.VMEM((16,), jnp.float32),
            pltpu.VMEM((16,), jnp.float32),
        ],
        compiler_params=pltpu.CompilerParams(use_tc_tiling_on_sc=False,
                                             needs_layout_passes=False),
    )(_seg_body)
    return f(err_lin, ids_pad, zeros1d, initc1d, sb16, mb16)


def _pack(x, dtype):
    xp = jnp.pad(x.astype(dtype), ((0, PAD_NODES), (0, 1)))
    return xp.reshape(R, C)


def kernel(pred_eps, true_eps, signs, free_mask, abs_mask, segment_ids):
    p4 = _pack(pred_eps, jnp.float32)
    t4 = _pack(true_eps, jnp.float32)
    tb4 = _pack(signs > 0, jnp.bool_)
    f4 = _pack(free_mask, jnp.bool_)
    a4 = _pack(abs_mask, jnp.bool_)

    err2d, bs2d, ms2d = _dense_call(p4, t4, tb4, f4, a4)
    err_lin = err2d.reshape(PN)

    ids_pad = jnp.concatenate(
        [segment_ids.astype(jnp.int32), jnp.zeros((PAD_NODES,), jnp.int32)])
    zeros1d = jnp.zeros((AW,), jnp.float32)
    initc1d = zeros1d.at[0].set(-float(PAD_NODES))
    sb16 = jnp.broadcast_to(bs2d[0, 0], (16,))
    mb16 = jnp.broadcast_to(ms2d[0, 0], (16,))

    return _seg_call(err_lin, ids_pad, zeros1d, initc1d, sb16, mb16)
